# Initial kernel scaffold; baseline (speedup 1.0000x reference)
import jax
import jax.numpy as jnp
from jax.experimental import pallas as pl


def kernel(fg_points, prj_points, x_lengths):
    raise NotImplementedError("write your pallas kernel here")


# speedup vs baseline: 1.7881x; 1.7881x over previous
import jax
import jax.numpy as jnp
from jax.experimental import pallas as pl
from jax.experimental.pallas import tpu as pltpu

_B, _N, _M = 8, 4096, 4096
_TILE_N = 256
_NT = _N // _TILE_N


def _chamfer_body(x_ref, yt_ref, out_ref, min_yx, acc):
    b = pl.program_id(0)
    j = pl.program_id(1)

    @pl.when(jnp.logical_and(b == 0, j == 0))
    def _():
        out_ref[0, 0] = 0.0

    @pl.when(j == 0)
    def _():
        min_yx[...] = jnp.full((1, _M), jnp.inf, jnp.float32)
        acc[0] = 0.0
        acc[1] = 0.0

    x = x_ref[0]
    yt = yt_ref[0]

    mask = ((x[:, 0:1] != 10000.0)
            & (x[:, 1:2] != 10000.0)
            & (x[:, 2:3] != 10000.0))

    d = jnp.zeros((_TILE_N, _M), jnp.float32)
    for k in range(3):
        diff = x[:, k:k + 1] - yt[k:k + 1, :]
        d = d + diff * diff

    min_xy = jnp.min(d, axis=1, keepdims=True)
    acc[0] += jnp.sum(jnp.where(mask, min_xy, 0.0))
    acc[1] += jnp.sum(mask.astype(jnp.float32))

    part = jnp.min(jnp.where(mask, d, jnp.inf), axis=0, keepdims=True)
    min_yx[...] = jnp.minimum(min_yx[...], part)

    @pl.when(j == _NT - 1)
    def _():
        loss_b = acc[0] / acc[1] + jnp.sum(min_yx[...]) / _M
        out_ref[0, 0] += loss_b / _B


@jax.jit
def _chamfer(fg, prj):
    yt = prj.transpose(0, 2, 1)
    out = pl.pallas_call(
        _chamfer_body,
        grid=(_B, _NT),
        in_specs=[
            pl.BlockSpec((1, _TILE_N, 3), lambda b, j: (b, j, 0)),
            pl.BlockSpec((1, 3, _M), lambda b, j: (b, 0, 0)),
        ],
        out_specs=pl.BlockSpec(memory_space=pltpu.SMEM),
        out_shape=jax.ShapeDtypeStruct((1, 1), jnp.float32),
        scratch_shapes=[
            pltpu.VMEM((1, _M), jnp.float32),
            pltpu.SMEM((2,), jnp.float32),
        ],
    )(fg, yt)
    return out[0, 0]


def kernel(fg_points, prj_points, x_lengths):
    del x_lengths
    return _chamfer(fg_points.astype(jnp.float32),
                    prj_points.astype(jnp.float32))
